# baseline (speedup 1.0000x reference)
import jax
import jax.numpy as jnp
from jax.experimental import pallas as pl
from jax.experimental.pallas import tpu as pltpu

K_CODES = 1024
LATENT_DIM = 256


def _vq_body(z_ref, w_ref, wn_ref, iota_ref, zq_ref, idx_ref, prob_ref):
    zt = z_ref[0]
    W = w_ref[...]
    k = W.shape[0]
    mm = jax.lax.dot_general(
        zt, W, (((0,), (1,)), ((), ())),
        preferred_element_type=jnp.float32,
    )
    zn_col = jnp.sum(zt * zt, axis=0)[:, None]
    wn_row = wn_ref[...]
    dist = (zn_col + wn_row) - 2.0 * mm
    rowmin = jnp.min(dist, axis=1, keepdims=True)
    e = jnp.exp(rowmin - dist)
    prob_ref[0] = e * (1.0 / jnp.sum(e, axis=1, keepdims=True))
    iota_row = iota_ref[...]
    masked = jnp.where(dist == rowmin, iota_row, float(k))
    idx_col = jnp.min(masked, axis=1, keepdims=True)
    idx_ref[0] = idx_col.astype(jnp.int32)
    onehot = jnp.where(iota_row == idx_col, 1.0, 0.0)
    zq_ref[0] = jax.lax.dot_general(
        W, onehot, (((0,), (1,)), ((), ())),
        preferred_element_type=jnp.float32,
    )


@jax.jit
def kernel(z, W):
    b, d, h, w = z.shape
    n = h * w
    z_r = z.reshape(b, d, n)
    wn2 = jnp.sum(W ** 2, axis=1)
    iota_row = jnp.arange(K_CODES, dtype=jnp.float32).reshape(1, K_CODES)
    zq, idx, prob = pl.pallas_call(
        _vq_body,
        grid=(b,),
        compiler_params=pltpu.CompilerParams(
            dimension_semantics=("parallel",)),
        in_specs=[
            pl.BlockSpec((1, d, n), lambda i: (i, 0, 0)),
            pl.BlockSpec((K_CODES, d), lambda i: (0, 0)),
            pl.BlockSpec((1, K_CODES), lambda i: (0, 0)),
            pl.BlockSpec((1, K_CODES), lambda i: (0, 0)),
        ],
        out_specs=[
            pl.BlockSpec((1, d, n), lambda i: (i, 0, 0)),
            pl.BlockSpec((1, n, 1), lambda i: (i, 0, 0)),
            pl.BlockSpec((1, n, K_CODES), lambda i: (i, 0, 0)),
        ],
        out_shape=[
            jax.ShapeDtypeStruct((b, d, n), jnp.float32),
            jax.ShapeDtypeStruct((b, n, 1), jnp.int32),
            jax.ShapeDtypeStruct((b, n, K_CODES), jnp.float32),
        ],
    )(z_r, W, wn2.reshape(1, K_CODES), iota_row)
    return (
        zq.reshape(b, d, h, w),
        idx.reshape(b * n),
        prob.reshape(b * n, K_CODES),
    )
